# baseline (speedup 1.0000x reference)
import functools

import jax
import jax.numpy as jnp
from jax import lax
from jax.experimental import pallas as pl
from jax.experimental.pallas import tpu as pltpu
from jax.experimental.pallas import tpu_sc as plsc

N_TOKENS = 16384
D_MODEL = 2048
N_EXPERTS = 64
BLOCK_M = 2048

NC = 2
NS = 16
L = 16
NW = NC * NS
ROWS_PER_W = N_TOKENS // NW
NGROUPS = ROWS_PER_W // L
NSTRIPE = 4
EPS = N_EXPERTS // NSTRIPE


def _merge_top2(a, b):
    m1a, i1a, m2a, i2a = a
    m1b, i1b, m2b, i2b = b
    b_wins = m1b > m1a
    m1 = jnp.where(b_wins, m1b, m1a)
    i1 = jnp.where(b_wins, i1b, i1a)
    ca = jnp.where(b_wins, m1a, m2a)
    ia = jnp.where(b_wins, i1a, i2a)
    cb = jnp.where(b_wins, m2b, m1b)
    ib = jnp.where(b_wins, i2b, i1b)
    b2_wins = cb > ca
    m2 = jnp.where(b2_wins, cb, ca)
    i2 = jnp.where(b2_wins, ib, ia)
    return m1, i1, m2, i2


def _sc_route_body(lt_hbm, probs_hbm, idx_hbm, sem, lt2_v, lt_v, probs_v, idx_v):
    wid = lax.axis_index("s") * NC + lax.axis_index("c")
    base = wid * ROWS_PER_W
    pltpu.async_copy(
        lt_hbm.at[pl.ds(wid * 2, 2), :], lt2_v, sem
    ).wait()

    zeros16 = jnp.zeros((L,), jnp.float32)

    def zero_blk(r, c):
        for j in range(32):
            probs_v[pl.ds(r * 512 + j * L, L)] = zeros16
        return c

    pass

    lane = lax.iota(jnp.int32, L)
    neg_inf = jnp.full((L,), -jnp.inf, jnp.float32)
    zeros_i = jnp.zeros((L,), jnp.int32)

    def group(g, c):
        off = g * L
        stripes = []
        for s in range(NSTRIPE):
            m1 = neg_inf
            i1 = zeros_i
            m2 = neg_inf
            i2 = zeros_i
            for e in range(s * EPS, (s + 1) * EPS):
                v = lt_v[e, pl.ds(off, L)]
                es = jnp.full((L,), e, jnp.int32)
                gt1 = v > m1
                gt2 = v > m2
                m2 = jnp.where(gt1, m1, jnp.where(gt2, v, m2))
                i2 = jnp.where(gt1, i1, jnp.where(gt2, es, i2))
                m1 = jnp.where(gt1, v, m1)
                i1 = jnp.where(gt1, es, i1)
            stripes.append((m1, i1, m2, i2))
        ab = _merge_top2(stripes[0], stripes[1])
        cd = _merge_top2(stripes[2], stripes[3])
        m1, i1, m2, i2 = _merge_top2(ab, cd)

        v1 = 1.0 / (1.0 + jnp.exp(m2 - m1))
        v2 = 1.0 - v1
        rows = off + lane
        plsc.store_scatter(probs_v, [rows * N_EXPERTS + i1], v1)
        plsc.store_scatter(probs_v, [rows * N_EXPERTS + i2], v2)
        plsc.store_scatter(idx_v, [rows * 2], i1)
        plsc.store_scatter(idx_v, [rows * 2 + 1], i2)
        return c

    pass

    pltpu.sync_copy(probs_v.at[pl.ds(0, 512)], probs_hbm.at[pl.ds(base * N_EXPERTS, 512)])
    pltpu.sync_copy(idx_v, idx_hbm.at[pl.ds(base * 2, ROWS_PER_W * 2)])


_sc_route = functools.partial(
    pl.kernel,
    out_type=[
        jax.ShapeDtypeStruct((N_TOKENS * N_EXPERTS,), jnp.float32),
        jax.ShapeDtypeStruct((N_TOKENS * 2,), jnp.int32),
    ],
    mesh=plsc.VectorSubcoreMesh(
        core_axis_name="c", subcore_axis_name="s", num_cores=NC, num_subcores=NS
    ),
    scratch_types=[
        pltpu.SemaphoreType.DMA,
        pltpu.VMEM((2, N_TOKENS), jnp.float32),
        pltpu.VMEM((N_EXPERTS, ROWS_PER_W), jnp.float32),
        pltpu.VMEM((ROWS_PER_W * N_EXPERTS,), jnp.float32),
        pltpu.VMEM((ROWS_PER_W * 2,), jnp.int32),
    ],
    compiler_params=pltpu.CompilerParams(needs_layout_passes=False),
)(_sc_route_body)


@jax.jit
def kernel(x, W):
    p, i = _sc_route(x.reshape(D_MODEL, N_TOKENS))
    return p.reshape(N_TOKENS, N_EXPERTS), i.reshape(N_TOKENS, 2)


# speedup vs baseline: 4.1737x; 4.1737x over previous
import jax
import jax.numpy as jnp
from jax import lax
from jax.experimental import pallas as pl

N_TOKENS = 16384
D_MODEL = 2048
N_EXPERTS = 64
BLOCK_M = 2048
IDX_ROWS = 8


def _router_body(x_ref, w_ref, probs_ref, idx_ref):
    x = x_ref[...]
    w = w_ref[...]
    lt = lax.dot_general(
        w, x, (((1,), (1,)), ((), ())), preferred_element_type=jnp.float32
    )
    row = lax.broadcasted_iota(jnp.int32, lt.shape, 0)

    m1 = jnp.max(lt, axis=0, keepdims=True)
    i1 = jnp.min(jnp.where(lt == m1, row, N_EXPERTS), axis=0, keepdims=True)
    masked = jnp.where(row == i1, -jnp.inf, lt)
    m2 = jnp.max(masked, axis=0, keepdims=True)
    i2 = jnp.min(jnp.where(masked == m2, row, N_EXPERTS), axis=0, keepdims=True)

    v1 = 1.0 / (1.0 + jnp.exp(m2 - m1))
    v2 = 1.0 - v1

    probs_ref[...] = jnp.where(
        row == i1, v1, jnp.where(row == i2, v2, jnp.float32(0.0))
    )
    rowp = lax.broadcasted_iota(jnp.int32, (IDX_ROWS, BLOCK_M), 0)
    idx_ref[...] = jnp.where(rowp == 0, i1, jnp.where(rowp == 1, i2, 0))


@jax.jit
def kernel(x, W):
    grid = (N_TOKENS // BLOCK_M,)
    probs_t, idx_t = pl.pallas_call(
        _router_body,
        grid=grid,
        in_specs=[
            pl.BlockSpec((BLOCK_M, D_MODEL), lambda i: (i, 0)),
            pl.BlockSpec((N_EXPERTS, D_MODEL), lambda i: (0, 0)),
        ],
        out_specs=[
            pl.BlockSpec((N_EXPERTS, BLOCK_M), lambda i: (0, i)),
            pl.BlockSpec((IDX_ROWS, BLOCK_M), lambda i: (0, i)),
        ],
        out_shape=[
            jax.ShapeDtypeStruct((N_EXPERTS, N_TOKENS), jnp.float32),
            jax.ShapeDtypeStruct((IDX_ROWS, N_TOKENS), jnp.int32),
        ],
    )(x, W)
    probs = probs_t.T
    idx = lax.slice(idx_t, (0, 0), (2, N_TOKENS)).T
    return probs, idx
